# Initial kernel scaffold; baseline (speedup 1.0000x reference)
import jax
import jax.numpy as jnp
from jax.experimental import pallas as pl


def kernel(x, node_attrs, edge_index, atomic_numbers, covalent_radii):
    raise NotImplementedError("write your pallas kernel here")


# speedup vs baseline: 233.2151x; 233.2151x over previous
import dataclasses
import functools

import jax
import jax.numpy as jnp
from jax import lax
from jax.experimental import pallas as pl
from jax.experimental.pallas import tpu as pltpu
from jax.experimental.pallas import tpu_sc as plsc

NUM_NODES = 100000
NUM_EDGES = 6400000
NUM_ELEM = 10
NW = 32
LANES = 16
NODES_PAD = 100352
NODES_PER_TILE = NODES_PAD // NW
EDGES_PER_TILE = NUM_EDGES // NW
CHUNK = 2000
N_CHUNKS = EDGES_PER_TILE // CHUNK

A_C = 1.0805
Q_C = 0.9183
P_C = 4.5791
QMP_C = Q_C - P_C
LN2 = 0.6931471805599453
SQRT2 = 1.4142135623730951

_mesh = plsc.VectorSubcoreMesh(core_axis_name="c", subcore_axis_name="s")

_cp = pltpu.CompilerParams()
if "needs_layout_passes" in pltpu.CompilerParams.__dataclass_fields__:
    _cp = dataclasses.replace(_cp, needs_layout_passes=False)


def _wid():
    return lax.axis_index("c") * 16 + lax.axis_index("s")


@functools.partial(
    pl.kernel,
    out_type=jax.ShapeDtypeStruct((NODES_PAD,), jnp.float32),
    mesh=_mesh,
    scratch_types=[
        pltpu.VMEM((NODES_PER_TILE * NUM_ELEM,), jnp.float32),
        pltpu.VMEM((LANES,), jnp.int32),
        pltpu.VMEM((128,), jnp.float32),
        pltpu.VMEM((NODES_PER_TILE,), jnp.float32),
    ],
    compiler_params=_cp,
)
def _node_kernel(attrs_hbm, anum_hbm, radii_hbm, rnode_hbm,
                 attrs_v, anum_v, radii_v, rnode_v):
    wid = _wid()
    base = wid * NODES_PER_TILE
    pltpu.sync_copy(attrs_hbm.at[pl.ds(base * NUM_ELEM, NODES_PER_TILE * NUM_ELEM)],
                    attrs_v)
    pltpu.sync_copy(anum_hbm, anum_v)
    pltpu.sync_copy(radii_hbm, radii_v)
    lanes = jnp.arange(LANES, dtype=jnp.int32)

    @pl.loop(0, NODES_PER_TILE, step=LANES)
    def _(g):
        idx0 = (g + lanes) * NUM_ELEM
        best = plsc.load_gather(attrs_v, [idx0])
        bidx = jnp.zeros((LANES,), jnp.int32)
        for j in range(1, NUM_ELEM):
            vj = plsc.load_gather(attrs_v, [idx0 + j])
            gt = vj > best
            best = jnp.where(gt, vj, best)
            bidx = jnp.where(gt, j, bidx)
        an = plsc.load_gather(anum_v, [bidx])
        r = plsc.load_gather(radii_v, [an])
        rnode_v[pl.ds(g, LANES)] = r

    pltpu.sync_copy(rnode_v, rnode_hbm.at[pl.ds(base, NODES_PER_TILE)])


@functools.partial(
    pl.kernel,
    out_type=jax.ShapeDtypeStruct((NUM_EDGES,), jnp.float32),
    mesh=_mesh,
    scratch_types=[
        pltpu.VMEM((NODES_PAD,), jnp.float32),
        pltpu.VMEM((CHUNK,), jnp.int32),
        pltpu.VMEM((CHUNK,), jnp.int32),
        pltpu.VMEM((CHUNK,), jnp.float32),
        pltpu.VMEM((CHUNK,), jnp.float32),
    ],
    compiler_params=_cp,
)
def _edge_kernel(rnode_hbm, s_hbm, r_hbm, x_hbm, out_hbm,
                 table_v, si_v, ri_v, x_v, o_v):
    wid = _wid()
    pltpu.sync_copy(rnode_hbm, table_v)
    ebase = wid * EDGES_PER_TILE

    @pl.loop(0, N_CHUNKS)
    def _(c):
        off = ebase + c * CHUNK
        pltpu.sync_copy(s_hbm.at[pl.ds(off, CHUNK)], si_v)
        pltpu.sync_copy(r_hbm.at[pl.ds(off, CHUNK)], ri_v)
        pltpu.sync_copy(x_hbm.at[pl.ds(off, CHUNK)], x_v)

        @pl.loop(0, CHUNK, step=LANES)
        def _(i):
            rs = plsc.load_gather(table_v, [si_v[pl.ds(i, LANES)]])
            rr = plsc.load_gather(table_v, [ri_v[pl.ds(i, LANES)]])
            xv = x_v[pl.ds(i, LANES)]
            t = xv / (0.5 * (rs + rr))
            bits = lax.bitcast_convert_type(t, jnp.int32)
            e = lax.shift_right_logical(bits, 23) - 127
            m = lax.bitcast_convert_type(
                (bits & 0x007FFFFF) | 0x3F800000, jnp.float32)
            big = m > SQRT2
            m = jnp.where(big, m * 0.5, m)
            e = e + jnp.where(big, 1, 0)
            z = (m - 1.0) / (m + 1.0)
            z2 = z * z
            lnm = 2.0 * z * (1.0 + z2 * (1.0 / 3.0 + z2 * (
                1.0 / 5.0 + z2 * (1.0 / 7.0 + z2 * (1.0 / 9.0)))))
            ln_t = e.astype(jnp.float32) * LN2 + lnm
            num = jnp.exp(Q_C * ln_t)
            den = jnp.exp(QMP_C * ln_t)
            o_v[pl.ds(i, LANES)] = (1.0 + den) / (1.0 + den + A_C * num)

        pltpu.sync_copy(o_v, out_hbm.at[pl.ds(off, CHUNK)])


def kernel(x, node_attrs, edge_index, atomic_numbers, covalent_radii):
    xf = x.reshape(-1)
    sender = edge_index[0]
    receiver = edge_index[1]
    attrs_flat = jnp.pad(
        node_attrs, ((0, NODES_PAD - NUM_NODES), (0, 0))).reshape(-1)
    anum_pad = jnp.pad(atomic_numbers, (0, LANES - NUM_ELEM))
    radii_pad = jnp.pad(covalent_radii, (0, 128 - covalent_radii.shape[0]))
    rnode = _node_kernel(attrs_flat, anum_pad, radii_pad)
    out = _edge_kernel(rnode, sender, receiver, xf)
    return out.reshape(NUM_EDGES, 1)


# speedup vs baseline: 280.1851x; 1.2014x over previous
import dataclasses
import functools

import jax
import jax.numpy as jnp
from jax import lax
from jax.experimental import pallas as pl
from jax.experimental.pallas import tpu as pltpu
from jax.experimental.pallas import tpu_sc as plsc

NUM_NODES = 100000
NUM_EDGES = 6400000
NUM_ELEM = 10
NW = 32
LANES = 16
NODES_PAD = 100352
NODES_PER_TILE = NODES_PAD // NW
EDGES_PER_TILE = NUM_EDGES // NW
CHUNK = 2000
N_CHUNKS = EDGES_PER_TILE // CHUNK

A_C = 1.0805
Q_C = 0.9183
P_C = 4.5791
QMP_C = Q_C - P_C
LN2 = 0.6931471805599453
SQRT2 = 1.4142135623730951

_mesh = plsc.VectorSubcoreMesh(core_axis_name="c", subcore_axis_name="s")

_cp = pltpu.CompilerParams()
if "needs_layout_passes" in pltpu.CompilerParams.__dataclass_fields__:
    _cp = dataclasses.replace(_cp, needs_layout_passes=False)


def _wid():
    return lax.axis_index("c") * 16 + lax.axis_index("s")


@functools.partial(
    pl.kernel,
    out_type=jax.ShapeDtypeStruct((NODES_PAD,), jnp.float32),
    mesh=_mesh,
    scratch_types=[
        pltpu.VMEM((NODES_PER_TILE * NUM_ELEM,), jnp.float32),
        pltpu.VMEM((LANES,), jnp.int32),
        pltpu.VMEM((128,), jnp.float32),
        pltpu.VMEM((NODES_PER_TILE,), jnp.float32),
    ],
    compiler_params=_cp,
)
def _node_kernel(attrs_hbm, anum_hbm, radii_hbm, rnode_hbm,
                 attrs_v, anum_v, radii_v, rnode_v):
    wid = _wid()
    base = wid * NODES_PER_TILE
    pltpu.sync_copy(attrs_hbm.at[pl.ds(base * NUM_ELEM, NODES_PER_TILE * NUM_ELEM)],
                    attrs_v)
    pltpu.sync_copy(anum_hbm, anum_v)
    pltpu.sync_copy(radii_hbm, radii_v)
    lanes = jnp.arange(LANES, dtype=jnp.int32)

    @pl.loop(0, NODES_PER_TILE, step=LANES)
    def _(g):
        idx0 = (g + lanes) * NUM_ELEM
        best = plsc.load_gather(attrs_v, [idx0])
        bidx = jnp.zeros((LANES,), jnp.int32)
        for j in range(1, NUM_ELEM):
            vj = plsc.load_gather(attrs_v, [idx0 + j])
            gt = vj > best
            best = jnp.where(gt, vj, best)
            bidx = jnp.where(gt, j, bidx)
        an = plsc.load_gather(anum_v, [bidx])
        r = plsc.load_gather(radii_v, [an])
        rnode_v[pl.ds(g, LANES)] = r

    pltpu.sync_copy(rnode_v, rnode_hbm.at[pl.ds(base, NODES_PER_TILE)])


UNROLL = 4


def _agnesi16(rs, rr, xv):
    t = (xv + xv) / (rs + rr)
    bits = lax.bitcast_convert_type(t, jnp.int32)
    e = lax.shift_right_logical(bits, 23) - 127
    m = lax.bitcast_convert_type(
        (bits & 0x007FFFFF) | 0x3F800000, jnp.float32)
    big = m > SQRT2
    m = jnp.where(big, m * 0.5, m)
    e = e + jnp.where(big, 1, 0)
    z = (m - 1.0) / (m + 1.0)
    z2 = z * z
    lnm = 2.0 * z * (1.0 + z2 * (1.0 / 3.0 + z2 * (
        1.0 / 5.0 + z2 * (1.0 / 7.0 + z2 * (1.0 / 9.0)))))
    ln_t = e.astype(jnp.float32) * LN2 + lnm
    num = jnp.exp(Q_C * ln_t)
    den1 = jnp.exp(QMP_C * ln_t) + 1.0
    return den1 / (den1 + A_C * num)


@functools.partial(
    pl.kernel,
    out_type=jax.ShapeDtypeStruct((NUM_EDGES,), jnp.float32),
    mesh=_mesh,
    scratch_types=[
        pltpu.VMEM((NODES_PAD,), jnp.float32),
        pltpu.VMEM((CHUNK,), jnp.int32),
        pltpu.VMEM((CHUNK,), jnp.int32),
        pltpu.VMEM((CHUNK,), jnp.float32),
        pltpu.VMEM((CHUNK,), jnp.float32),
        pltpu.VMEM((CHUNK,), jnp.int32),
        pltpu.VMEM((CHUNK,), jnp.int32),
        pltpu.VMEM((CHUNK,), jnp.float32),
        pltpu.VMEM((CHUNK,), jnp.float32),
        pltpu.SemaphoreType.DMA,
        pltpu.SemaphoreType.DMA,
        pltpu.SemaphoreType.DMA,
        pltpu.SemaphoreType.DMA,
    ],
    compiler_params=_cp,
)
def _edge_kernel(rnode_hbm, s_hbm, r_hbm, x_hbm, out_hbm,
                 table_v, si_a, ri_a, x_a, o_a, si_b, ri_b, x_b, o_b,
                 sin_a, sin_b, sout_a, sout_b):
    wid = _wid()
    pltpu.sync_copy(rnode_hbm, table_v)
    ebase = wid * EDGES_PER_TILE

    def start_in(c, si_v, ri_v, x_v, sem):
        off = ebase + c * CHUNK
        pltpu.async_copy(s_hbm.at[pl.ds(off, CHUNK)], si_v, sem)
        pltpu.async_copy(r_hbm.at[pl.ds(off, CHUNK)], ri_v, sem)
        pltpu.async_copy(x_hbm.at[pl.ds(off, CHUNK)], x_v, sem)

    def wait_in(si_v, ri_v, x_v, sem):
        pltpu.make_async_copy(s_hbm.at[pl.ds(ebase, CHUNK)], si_v, sem).wait()
        pltpu.make_async_copy(r_hbm.at[pl.ds(ebase, CHUNK)], ri_v, sem).wait()
        pltpu.make_async_copy(x_hbm.at[pl.ds(ebase, CHUNK)], x_v, sem).wait()

    def start_out(c, o_v, sem):
        off = ebase + c * CHUNK
        pltpu.async_copy(o_v, out_hbm.at[pl.ds(off, CHUNK)], sem)

    def wait_out(o_v, sem):
        pltpu.make_async_copy(o_v, out_hbm.at[pl.ds(ebase, CHUNK)], sem).wait()

    def compute(si_v, ri_v, x_v, o_v):
        @pl.loop(0, CHUNK, step=UNROLL * LANES)
        def _(i):
            for u in range(UNROLL):
                ii = i + u * LANES
                rs = plsc.load_gather(table_v, [si_v[pl.ds(ii, LANES)]])
                rr = plsc.load_gather(table_v, [ri_v[pl.ds(ii, LANES)]])
                o_v[pl.ds(ii, LANES)] = _agnesi16(rs, rr, x_v[pl.ds(ii, LANES)])

    start_in(0, si_a, ri_a, x_a, sin_a)
    start_in(1, si_b, ri_b, x_b, sin_b)

    wait_in(si_a, ri_a, x_a, sin_a)
    compute(si_a, ri_a, x_a, o_a)
    start_out(0, o_a, sout_a)
    start_in(2, si_a, ri_a, x_a, sin_a)

    wait_in(si_b, ri_b, x_b, sin_b)
    compute(si_b, ri_b, x_b, o_b)
    start_out(1, o_b, sout_b)
    start_in(3, si_b, ri_b, x_b, sin_b)

    @pl.loop(1, N_CHUNKS // 2 - 1)
    def _(k):
        c0 = 2 * k

        wait_in(si_a, ri_a, x_a, sin_a)
        wait_out(o_a, sout_a)
        compute(si_a, ri_a, x_a, o_a)
        start_out(c0, o_a, sout_a)
        start_in(c0 + 2, si_a, ri_a, x_a, sin_a)

        wait_in(si_b, ri_b, x_b, sin_b)
        wait_out(o_b, sout_b)
        compute(si_b, ri_b, x_b, o_b)
        start_out(c0 + 1, o_b, sout_b)
        start_in(c0 + 3, si_b, ri_b, x_b, sin_b)

    wait_in(si_a, ri_a, x_a, sin_a)
    wait_out(o_a, sout_a)
    compute(si_a, ri_a, x_a, o_a)
    start_out(N_CHUNKS - 2, o_a, sout_a)

    wait_in(si_b, ri_b, x_b, sin_b)
    wait_out(o_b, sout_b)
    compute(si_b, ri_b, x_b, o_b)
    start_out(N_CHUNKS - 1, o_b, sout_b)

    wait_out(o_a, sout_a)
    wait_out(o_b, sout_b)


def kernel(x, node_attrs, edge_index, atomic_numbers, covalent_radii):
    xf = x.reshape(-1)
    sender = edge_index[0]
    receiver = edge_index[1]
    attrs_flat = jnp.pad(
        node_attrs, ((0, NODES_PAD - NUM_NODES), (0, 0))).reshape(-1)
    anum_pad = jnp.pad(atomic_numbers, (0, LANES - NUM_ELEM))
    radii_pad = jnp.pad(covalent_radii, (0, 128 - covalent_radii.shape[0]))
    rnode = _node_kernel(attrs_flat, anum_pad, radii_pad)
    out = _edge_kernel(rnode, sender, receiver, xf)
    return out.reshape(NUM_EDGES, 1)
